# baseline (speedup 1.0000x reference)
import functools

import jax
import jax.numpy as jnp
from jax import lax
from jax.experimental import pallas as pl
from jax.experimental.pallas import tpu as pltpu
from jax.experimental.pallas import tpu_sc as plsc

N = 10000
E = 160000
H = 256

NC = 2
NS = 16
NW = NC * NS

P = 5
EP = E // P

G_CHUNK = 40
G_NCHUNK = (EP // NW) // G_CHUNK

S_PER_TILE = EP // NS
S_CHUNK = 80
S_NCHUNK = S_PER_TILE // S_CHUNK

N_PAD = 10240


def _ln(x, g, b):
    mu = jnp.mean(x, axis=-1, keepdims=True)
    xc = x - mu
    var = jnp.mean(xc * xc, axis=-1, keepdims=True)
    return xc * lax.rsqrt(var + 1e-5) * g + b


def _pack_bf16_pair(x):
    xb = x.astype(jnp.bfloat16)
    lo = lax.bitcast_convert_type(xb[:, :128], jnp.uint16).astype(jnp.uint32)
    hi = lax.bitcast_convert_type(xb[:, 128:], jnp.uint16).astype(jnp.uint32)
    return lax.bitcast_convert_type(lo | (hi << 16), jnp.int32)


def _unpack_bf16_pair(w):
    lo = lax.bitcast_convert_type(w << 16, jnp.float32)
    hi = lax.bitcast_convert_type(w & jnp.int32(-65536), jnp.float32)
    return lo, hi


def _proj_body(nf, w_s, w_r, b0, ps, pr):
    x = nf[...]
    s = jnp.dot(x, w_s[...], preferred_element_type=jnp.float32)
    r = jnp.dot(x, w_r[...], preferred_element_type=jnp.float32) + b0[...]
    ps[...] = _pack_bf16_pair(s)
    pr[...] = _pack_bf16_pair(r)


def _edge_body(*refs, n_extra=0):
    gs, gr, ef, w_e, w1, w2, b1, b2, g, bt = refs[:10]
    new_edge, pre_t = refs[10 + n_extra:]
    e = ef[...]
    s_lo, s_hi = _unpack_bf16_pair(gs[...])
    r_lo, r_hi = _unpack_bf16_pair(gr[...])
    gsum = jnp.concatenate([s_lo + r_lo, s_hi + r_hi], axis=1)
    x = gsum + jnp.dot(e, w_e[...], preferred_element_type=jnp.float32)
    x = jnp.maximum(x, 0.0)
    x = jnp.dot(x, w1[...], preferred_element_type=jnp.float32) + b1[...]
    x = jnp.maximum(x, 0.0)
    x = jnp.dot(x, w2[...], preferred_element_type=jnp.float32) + b2[...]
    y = _ln(x, g[...], bt[...])
    new_edge[...] = y + e
    pre_t[0] = y[:, :128]
    pre_t[1] = y[:, 128:]


def _node_body(nf, *rest):
    aggs = rest[:2 * P]
    w0, w0lo, w0hi, w1, w2, b0, b1, b2, g, bt = rest[2 * P:-1]
    out = rest[-1]
    x0 = nf[...]
    alo = aggs[0][0]
    ahi = aggs[1][0]
    for p in range(1, P):
        alo = alo + aggs[2 * p][0]
        ahi = ahi + aggs[2 * p + 1][0]
    x = (jnp.dot(x0, w0[...], preferred_element_type=jnp.float32)
         + jnp.dot(alo, w0lo[...], preferred_element_type=jnp.float32)
         + jnp.dot(ahi, w0hi[...], preferred_element_type=jnp.float32)
         + b0[...])
    x = jnp.maximum(x, 0.0)
    x = jnp.dot(x, w1[...], preferred_element_type=jnp.float32) + b1[...]
    x = jnp.maximum(x, 0.0)
    x = jnp.dot(x, w2[...], preferred_element_type=jnp.float32) + b2[...]
    out[...] = _ln(x, g[...], bt[...]) + x0


def _full(shape):
    return pl.BlockSpec(shape, lambda i: (0,) * len(shape))


def _rows(bm, w):
    return pl.BlockSpec((bm, w), lambda i: (i, 0))


@functools.cache
def _sc_kernels():
    mesh = plsc.VectorSubcoreMesh(
        core_axis_name="c", subcore_axis_name="s",
        num_cores=NC, num_subcores=NS)

    @functools.partial(
        pl.kernel,
        out_type=[jax.ShapeDtypeStruct((NW, G_NCHUNK, G_CHUNK, H // 2),
                                       jnp.int32)] * 2,
        mesh=mesh,
        scratch_types=[
            pltpu.VMEM((G_NCHUNK, G_CHUNK), jnp.int32),
            pltpu.VMEM((G_NCHUNK, G_CHUNK), jnp.int32),
            [pltpu.VMEM((G_CHUNK, H // 2), jnp.int32)] * 2,
            [pltpu.VMEM((G_CHUNK, H // 2), jnp.int32)] * 2,
            [pltpu.SemaphoreType.DMA] * 2,
            [pltpu.SemaphoreType.DMA] * 2,
            [pltpu.SemaphoreType.DMA] * 2,
            [pltpu.SemaphoreType.DMA] * 2,
        ],
    )
    def sc_gather(ps_hbm, pr_hbm, sidx_hbm, ridx_hbm, outs_hbm, outr_hbm,
                  sidx_v, ridx_v, rows_a, rows_b, sem_a, sem_b, sem_ws,
                  sem_wr):
        wid = lax.axis_index("s") * NC + lax.axis_index("c")
        pltpu.sync_copy(sidx_hbm.at[wid], sidx_v)
        pltpu.sync_copy(ridx_hbm.at[wid], ridx_v)

        def start(j, b):
            pltpu.async_copy(ps_hbm.at[sidx_v.at[j]], rows_a[b], sem_a[b])
            pltpu.async_copy(pr_hbm.at[ridx_v.at[j]], rows_b[b], sem_b[b])

        def process(j, b):
            pltpu.make_async_copy(ps_hbm.at[sidx_v.at[j]], rows_a[b],
                                  sem_a[b]).wait()
            pltpu.make_async_copy(pr_hbm.at[ridx_v.at[j]], rows_b[b],
                                  sem_b[b]).wait()
            pltpu.async_copy(rows_a[b], outs_hbm.at[wid, j], sem_ws[b])
            pltpu.async_copy(rows_b[b], outr_hbm.at[wid, j], sem_wr[b])

        def wait_write(j, b):
            pltpu.make_async_copy(rows_a[b], outs_hbm.at[wid, j],
                                  sem_ws[b]).wait()
            pltpu.make_async_copy(rows_b[b], outr_hbm.at[wid, j],
                                  sem_wr[b]).wait()

        start(0, 0)

        def pair(t, carry):
            for b in range(2):
                j = 2 * t + b

                @pl.when(j > 0)
                def _():
                    wait_write(j - 1, 1 - b)

                start(j + 1, 1 - b)
                process(j, b)
            return carry

        lax.fori_loop(0, (G_NCHUNK - 1) // 2, pair, 0, unroll=1)
        wait_write(G_NCHUNK - 2, 1)
        process(G_NCHUNK - 1, 0)
        wait_write(G_NCHUNK - 1, 0)

    @functools.partial(
        pl.kernel,
        out_type=jax.ShapeDtypeStruct((NC, N_PAD, 128), jnp.float32),
        mesh=mesh,
        scratch_types=[
            pltpu.VMEM((S_NCHUNK, S_CHUNK), jnp.int32),
            [pltpu.VMEM((S_CHUNK, 128), jnp.float32)] * 2,
            pltpu.VMEM_SHARED((N_PAD, 128), jnp.float32),
            [pltpu.SemaphoreType.DMA] * 2,
            [pltpu.SemaphoreType.DMA] * 2,
        ],
    )
    def sc_scatter(pre_hbm, ridx_hbm, zeros_hbm, out_hbm,
                   ridx_v, rows_v, acc, sem_l, sem_s):
        c = lax.axis_index("c")
        s = lax.axis_index("s")
        pltpu.sync_copy(zeros_hbm, acc.at[pl.ds(s * (N_PAD // NS), N_PAD // NS)])
        plsc.subcore_barrier()
        pltpu.sync_copy(ridx_hbm.at[s], ridx_v)

        def _src(j):
            return pre_hbm.at[c, pl.ds(s * S_PER_TILE + j * S_CHUNK, S_CHUNK)]

        def start_load(j, b):
            pltpu.async_copy(_src(j), rows_v[b], sem_l[b])

        def start_scatter(j, b):
            pltpu.async_copy(rows_v[b], acc.at[ridx_v.at[j]], sem_s[b],
                             add=True)

        def wait_load(j, b):
            pltpu.make_async_copy(_src(j), rows_v[b], sem_l[b]).wait()

        def wait_scatter(j, b):
            pltpu.make_async_copy(rows_v[b], acc.at[ridx_v.at[j]],
                                  sem_s[b]).wait()

        start_load(0, 0)

        def pair(t, carry):
            for b in range(2):
                j = 2 * t + b

                @pl.when(j > 0)
                def _():
                    wait_scatter(j - 1, 1 - b)

                start_load(j + 1, 1 - b)
                wait_load(j, b)
                start_scatter(j, b)
            return carry

        lax.fori_loop(0, (S_NCHUNK - 1) // 2, pair, 0, unroll=1)
        wait_scatter(S_NCHUNK - 2, 1)
        wait_load(S_NCHUNK - 1, 0)
        start_scatter(S_NCHUNK - 1, 0)
        wait_scatter(S_NCHUNK - 1, 0)
        plsc.subcore_barrier()
        rpt = N_PAD // NS
        pltpu.sync_copy(acc.at[pl.ds(s * rpt, rpt)],
                        out_hbm.at[c, pl.ds(s * rpt, rpt)])

    return sc_gather, sc_scatter


def _sc_gather(ps, pr, sidx, ridx):
    return _sc_kernels()[0](ps, pr, sidx, ridx)


def _sc_scatter(pre_t, ridx_t, zeros):
    return _sc_kernels()[1](pre_t, ridx_t, zeros)


def kernel(senders, receivers, node_features, edge_features,
           eW0, eb0, eW1, eb1, eW2, eb2, eg, ebt,
           nW0, nb0, nW1, nb1, nW2, nb2, ng, nbt):
    f32 = jnp.float32
    nf = node_features
    ef = edge_features

    eb0r = eb0.reshape(1, H)
    eb1r = eb1.reshape(1, H)
    eb2r = eb2.reshape(1, H)
    egr = eg.reshape(1, H)
    ebtr = ebt.reshape(1, H)
    nb0r = nb0.reshape(1, H)
    nb1r = nb1.reshape(1, H)
    nb2r = nb2.reshape(1, H)
    ngr = ng.reshape(1, H)
    nbtr = nbt.reshape(1, H)

    BN = 2000
    ps32, pr32 = pl.pallas_call(
        _proj_body,
        grid=(N // BN,),
        in_specs=[_rows(BN, H), _full((H, H)), _full((H, H)), _full((1, H))],
        out_specs=[_rows(BN, H // 2)] * 2,
        out_shape=[jax.ShapeDtypeStruct((N, H // 2), jnp.int32)] * 2,
    )(nf, eW0[:H], eW0[H:2 * H], eb0r)

    BE = 2000
    nbp = EP // BE
    sidx = senders.reshape(P, NW, G_NCHUNK, G_CHUNK)
    ridx = receivers.reshape(P, NW, G_NCHUNK, G_CHUNK)
    ridx_t = receivers.reshape(P, NS, S_NCHUNK, S_CHUNK)
    zeros = jnp.zeros((N_PAD // NS, 128), f32)

    new_edge = None
    aggs = []
    for p in range(P):
        gs32, gr32 = _sc_gather(ps32, pr32, sidx[p], ridx[p])
        gs = gs32.reshape(EP, H // 2)
        gr = gr32.reshape(EP, H // 2)

        def _off(i, p=p):
            return (p * nbp + i, 0)

        in_specs = [_rows(BE, H // 2), _rows(BE, H // 2),
                    pl.BlockSpec((BE, H), _off),
                    _full((H, H)), _full((H, H)), _full((H, H)),
                    _full((1, H)), _full((1, H)), _full((1, H)),
                    _full((1, H))]
        ins = [gs, gr, ef, eW0[2 * H:], eW1, eW2, eb1r, eb2r, egr, ebtr]
        aliases = {}
        if p > 0:
            in_specs.append(pl.BlockSpec(memory_space=pl.ANY))
            ins.append(new_edge)
            aliases = {len(ins) - 1: 0}
        new_edge, pre_p = pl.pallas_call(
            functools.partial(_edge_body, n_extra=len(aliases)),
            grid=(nbp,),
            in_specs=in_specs,
            out_specs=[pl.BlockSpec((BE, H), _off),
                       pl.BlockSpec((2, BE, 128), lambda i: (0, i, 0))],
            out_shape=[jax.ShapeDtypeStruct((E, H), f32),
                       jax.ShapeDtypeStruct((2, EP, 128), f32)],
            input_output_aliases=aliases,
        )(*ins)
        aggs.append(_sc_scatter(pre_p, ridx_t[p], zeros))

    agg_specs = []
    agg_ins = []
    for p in range(P):
        agg_specs.append(pl.BlockSpec((1, BN, 128), lambda i: (0, i, 0)))
        agg_specs.append(pl.BlockSpec((1, BN, 128), lambda i: (1, i, 0)))
        agg_ins.extend([aggs[p], aggs[p]])
    new_node = pl.pallas_call(
        _node_body,
        grid=(N // BN,),
        in_specs=[_rows(BN, H)] + agg_specs
                 + [_full((H, H)), _full((128, H)), _full((128, H)),
                    _full((H, H)), _full((H, H)),
                    _full((1, H)), _full((1, H)), _full((1, H)),
                    _full((1, H)), _full((1, H))],
        out_specs=_rows(BN, H),
        out_shape=jax.ShapeDtypeStruct((N, H), f32),
    )(nf, *agg_ins, nW0[:H], nW0[H:H + 128], nW0[H + 128:],
      nW1, nW2, nb0r, nb1r, nb2r, ngr, nbtr)

    return (new_node, new_edge)


# speedup vs baseline: 1.0813x; 1.0813x over previous
import functools

import jax
import jax.numpy as jnp
from jax import lax
from jax.experimental import pallas as pl
from jax.experimental.pallas import tpu as pltpu
from jax.experimental.pallas import tpu_sc as plsc

N = 10000
E = 160000
H = 256

NC = 2
NS = 16
NW = NC * NS

P = 5
EP = E // P

G_CHUNK = 40
G_NCHUNK = (EP // NW) // G_CHUNK

S_PER_TILE = E // NS
S_CHUNK = 80
S_NCHUNK = S_PER_TILE // S_CHUNK

N_PAD = 10240


def _ln(x, g, b):
    mu = jnp.mean(x, axis=-1, keepdims=True)
    xc = x - mu
    var = jnp.mean(xc * xc, axis=-1, keepdims=True)
    return xc * lax.rsqrt(var + 1e-5) * g + b


def _pack_bf16_pair(x):
    xb = x.astype(jnp.bfloat16)
    lo = lax.bitcast_convert_type(xb[:, :128], jnp.uint16).astype(jnp.uint32)
    hi = lax.bitcast_convert_type(xb[:, 128:], jnp.uint16).astype(jnp.uint32)
    return lax.bitcast_convert_type(lo | (hi << 16), jnp.int32)


def _unpack_bf16_pair(w):
    lo = lax.bitcast_convert_type(w << 16, jnp.float32)
    hi = lax.bitcast_convert_type(w & jnp.int32(-65536), jnp.float32)
    return lo, hi


def _proj_body(nf, w_s, w_r, b0, ps, pr):
    x = nf[...]
    s = jnp.dot(x, w_s[...], preferred_element_type=jnp.float32)
    r = jnp.dot(x, w_r[...], preferred_element_type=jnp.float32) + b0[...]
    ps[...] = _pack_bf16_pair(s)
    pr[...] = _pack_bf16_pair(r)


def _edge_body(*refs, n_extra=0):
    gs, gr, ef, w_e, w1, w2, b1, b2, g, bt = refs[:10]
    new_edge, pre_t = refs[10 + n_extra:]
    e = ef[...]
    s_lo, s_hi = _unpack_bf16_pair(gs[...])
    r_lo, r_hi = _unpack_bf16_pair(gr[...])
    gsum = jnp.concatenate([s_lo + r_lo, s_hi + r_hi], axis=1)
    x = gsum + jnp.dot(e, w_e[...], preferred_element_type=jnp.float32)
    x = jnp.maximum(x, 0.0)
    x = jnp.dot(x, w1[...], preferred_element_type=jnp.float32) + b1[...]
    x = jnp.maximum(x, 0.0)
    x = jnp.dot(x, w2[...], preferred_element_type=jnp.float32) + b2[...]
    y = _ln(x, g[...], bt[...])
    new_edge[...] = y + e
    pre_t[0] = y[:, :128]
    pre_t[1] = y[:, 128:]


def _node_body(nf, a0, a1, w0, w0lo, w0hi, w1, w2, b0, b1, b2, g, bt, out):
    x0 = nf[...]
    alo = a0[0]
    ahi = a1[0]
    x = (jnp.dot(x0, w0[...], preferred_element_type=jnp.float32)
         + jnp.dot(alo, w0lo[...], preferred_element_type=jnp.float32)
         + jnp.dot(ahi, w0hi[...], preferred_element_type=jnp.float32)
         + b0[...])
    x = jnp.maximum(x, 0.0)
    x = jnp.dot(x, w1[...], preferred_element_type=jnp.float32) + b1[...]
    x = jnp.maximum(x, 0.0)
    x = jnp.dot(x, w2[...], preferred_element_type=jnp.float32) + b2[...]
    out[...] = _ln(x, g[...], bt[...]) + x0


def _full(shape):
    return pl.BlockSpec(shape, lambda i: (0,) * len(shape))


def _rows(bm, w):
    return pl.BlockSpec((bm, w), lambda i: (i, 0))


@functools.cache
def _sc_kernels():
    mesh = plsc.VectorSubcoreMesh(
        core_axis_name="c", subcore_axis_name="s",
        num_cores=NC, num_subcores=NS)

    @functools.partial(
        pl.kernel,
        out_type=[jax.ShapeDtypeStruct((NW, G_NCHUNK, G_CHUNK, H // 2),
                                       jnp.int32)] * 2,
        mesh=mesh,
        scratch_types=[
            pltpu.VMEM((G_NCHUNK, G_CHUNK), jnp.int32),
            pltpu.VMEM((G_NCHUNK, G_CHUNK), jnp.int32),
            [pltpu.VMEM((G_CHUNK, H // 2), jnp.int32)] * 2,
            [pltpu.VMEM((G_CHUNK, H // 2), jnp.int32)] * 2,
            [pltpu.SemaphoreType.DMA] * 2,
            [pltpu.SemaphoreType.DMA] * 2,
            [pltpu.SemaphoreType.DMA] * 2,
            [pltpu.SemaphoreType.DMA] * 2,
        ],
    )
    def sc_gather(ps_hbm, pr_hbm, sidx_hbm, ridx_hbm, outs_hbm, outr_hbm,
                  sidx_v, ridx_v, rows_a, rows_b, sem_a, sem_b, sem_ws,
                  sem_wr):
        wid = lax.axis_index("s") * NC + lax.axis_index("c")
        pltpu.sync_copy(sidx_hbm.at[wid], sidx_v)
        pltpu.sync_copy(ridx_hbm.at[wid], ridx_v)

        def start(j, b):
            pltpu.async_copy(ps_hbm.at[sidx_v.at[j]], rows_a[b], sem_a[b])
            pltpu.async_copy(pr_hbm.at[ridx_v.at[j]], rows_b[b], sem_b[b])

        def process(j, b):
            pltpu.make_async_copy(ps_hbm.at[sidx_v.at[j]], rows_a[b],
                                  sem_a[b]).wait()
            pltpu.make_async_copy(pr_hbm.at[ridx_v.at[j]], rows_b[b],
                                  sem_b[b]).wait()
            pltpu.async_copy(rows_a[b], outs_hbm.at[wid, j], sem_ws[b])
            pltpu.async_copy(rows_b[b], outr_hbm.at[wid, j], sem_wr[b])

        def wait_write(j, b):
            pltpu.make_async_copy(rows_a[b], outs_hbm.at[wid, j],
                                  sem_ws[b]).wait()
            pltpu.make_async_copy(rows_b[b], outr_hbm.at[wid, j],
                                  sem_wr[b]).wait()

        start(0, 0)

        def pair(t, carry):
            for b in range(2):
                j = 2 * t + b

                @pl.when(j > 0)
                def _():
                    wait_write(j - 1, 1 - b)

                start(j + 1, 1 - b)
                process(j, b)
            return carry

        lax.fori_loop(0, (G_NCHUNK - 1) // 2, pair, 0, unroll=1)
        wait_write(G_NCHUNK - 2, 1)
        process(G_NCHUNK - 1, 0)
        wait_write(G_NCHUNK - 1, 0)

    @functools.partial(
        pl.kernel,
        out_type=jax.ShapeDtypeStruct((NC, N_PAD, 128), jnp.float32),
        mesh=mesh,
        scratch_types=[
            pltpu.VMEM((S_NCHUNK, S_CHUNK), jnp.int32),
            [pltpu.VMEM((S_CHUNK, 128), jnp.float32)] * 2,
            pltpu.VMEM_SHARED((N_PAD, 128), jnp.float32),
            [pltpu.SemaphoreType.DMA] * 2,
            [pltpu.SemaphoreType.DMA] * 2,
        ],
    )
    def sc_scatter(pre_hbm, ridx_hbm, zeros_hbm, out_hbm,
                   ridx_v, rows_v, acc, sem_l, sem_s):
        c = lax.axis_index("c")
        s = lax.axis_index("s")
        pltpu.sync_copy(zeros_hbm, acc.at[pl.ds(s * (N_PAD // NS), N_PAD // NS)])
        plsc.subcore_barrier()
        pltpu.sync_copy(ridx_hbm.at[s], ridx_v)

        def _src(j):
            return pre_hbm.at[c, pl.ds(s * S_PER_TILE + j * S_CHUNK, S_CHUNK)]

        def start_load(j, b):
            pltpu.async_copy(_src(j), rows_v[b], sem_l[b])

        def start_scatter(j, b):
            pltpu.async_copy(rows_v[b], acc.at[ridx_v.at[j]], sem_s[b],
                             add=True)

        def wait_load(j, b):
            pltpu.make_async_copy(_src(j), rows_v[b], sem_l[b]).wait()

        def wait_scatter(j, b):
            pltpu.make_async_copy(rows_v[b], acc.at[ridx_v.at[j]],
                                  sem_s[b]).wait()

        start_load(0, 0)

        def pair(t, carry):
            for b in range(2):
                j = 2 * t + b

                @pl.when(j > 0)
                def _():
                    wait_scatter(j - 1, 1 - b)

                start_load(j + 1, 1 - b)
                wait_load(j, b)
                start_scatter(j, b)
            return carry

        lax.fori_loop(0, (S_NCHUNK - 1) // 2, pair, 0, unroll=1)
        wait_scatter(S_NCHUNK - 2, 1)
        wait_load(S_NCHUNK - 1, 0)
        start_scatter(S_NCHUNK - 1, 0)
        wait_scatter(S_NCHUNK - 1, 0)
        plsc.subcore_barrier()
        rpt = N_PAD // NS
        pltpu.sync_copy(acc.at[pl.ds(s * rpt, rpt)],
                        out_hbm.at[c, pl.ds(s * rpt, rpt)])

    return sc_gather, sc_scatter


def _sc_gather(ps, pr, sidx, ridx):
    return _sc_kernels()[0](ps, pr, sidx, ridx)


def _sc_scatter(pre_t, ridx_t, zeros):
    return _sc_kernels()[1](pre_t, ridx_t, zeros)


def kernel(senders, receivers, node_features, edge_features,
           eW0, eb0, eW1, eb1, eW2, eb2, eg, ebt,
           nW0, nb0, nW1, nb1, nW2, nb2, ng, nbt):
    f32 = jnp.float32
    nf = node_features
    ef = edge_features

    eb0r = eb0.reshape(1, H)
    eb1r = eb1.reshape(1, H)
    eb2r = eb2.reshape(1, H)
    egr = eg.reshape(1, H)
    ebtr = ebt.reshape(1, H)
    nb0r = nb0.reshape(1, H)
    nb1r = nb1.reshape(1, H)
    nb2r = nb2.reshape(1, H)
    ngr = ng.reshape(1, H)
    nbtr = nbt.reshape(1, H)

    BN = 2000
    ps32, pr32 = pl.pallas_call(
        _proj_body,
        grid=(N // BN,),
        in_specs=[_rows(BN, H), _full((H, H)), _full((H, H)), _full((1, H))],
        out_specs=[_rows(BN, H // 2)] * 2,
        out_shape=[jax.ShapeDtypeStruct((N, H // 2), jnp.int32)] * 2,
    )(nf, eW0[:H], eW0[H:2 * H], eb0r)

    BE = 2000
    nbp = EP // BE
    sidx = senders.reshape(P, NW, G_NCHUNK, G_CHUNK)
    ridx = receivers.reshape(P, NW, G_NCHUNK, G_CHUNK)
    zeros = jnp.zeros((N_PAD // NS, 128), f32)

    new_edge = None
    pre_t = None
    for p in range(P):
        gs32, gr32 = _sc_gather(ps32, pr32, sidx[p], ridx[p])
        gs = gs32.reshape(EP, H // 2)
        gr = gr32.reshape(EP, H // 2)

        def _off(i, p=p):
            return (p * nbp + i, 0)

        def _off3(i, p=p):
            return (0, p * nbp + i, 0)

        in_specs = [_rows(BE, H // 2), _rows(BE, H // 2),
                    pl.BlockSpec((BE, H), _off),
                    _full((H, H)), _full((H, H)), _full((H, H)),
                    _full((1, H)), _full((1, H)), _full((1, H)),
                    _full((1, H))]
        ins = [gs, gr, ef, eW0[2 * H:], eW1, eW2, eb1r, eb2r, egr, ebtr]
        aliases = {}
        if p > 0:
            in_specs.append(pl.BlockSpec(memory_space=pl.ANY))
            ins.append(new_edge)
            aliases[len(ins) - 1] = 0
            in_specs.append(pl.BlockSpec(memory_space=pl.ANY))
            ins.append(pre_t)
            aliases[len(ins) - 1] = 1
        new_edge, pre_t = pl.pallas_call(
            functools.partial(_edge_body, n_extra=len(aliases)),
            grid=(nbp,),
            in_specs=in_specs,
            out_specs=[pl.BlockSpec((BE, H), _off),
                       pl.BlockSpec((2, BE, 128), _off3)],
            out_shape=[jax.ShapeDtypeStruct((E, H), f32),
                       jax.ShapeDtypeStruct((2, E, 128), f32)],
            input_output_aliases=aliases,
        )(*ins)

    ridx_t = receivers.reshape(NS, S_NCHUNK, S_CHUNK)
    agg_t = _sc_scatter(pre_t, ridx_t, zeros)

    new_node = pl.pallas_call(
        _node_body,
        grid=(N // BN,),
        in_specs=[_rows(BN, H),
                  pl.BlockSpec((1, BN, 128), lambda i: (0, i, 0)),
                  pl.BlockSpec((1, BN, 128), lambda i: (1, i, 0)),
                  _full((H, H)), _full((128, H)), _full((128, H)),
                  _full((H, H)), _full((H, H)),
                  _full((1, H)), _full((1, H)), _full((1, H)),
                  _full((1, H)), _full((1, H))],
        out_specs=_rows(BN, H),
        out_shape=jax.ShapeDtypeStruct((N, H), f32),
    )(nf, agg_t, agg_t, nW0[:H], nW0[H:H + 128], nW0[H + 128:],
      nW1, nW2, nb0r, nb1r, nb2r, ngr, nbtr)

    return (new_node, new_edge)
